# baseline (speedup 1.0000x reference)
import functools

import jax
import jax.numpy as jnp
from jax.experimental import pallas as pl
from jax.experimental.pallas import tpu as pltpu

_K = 20
_CHUNK = 128
_GW = 128


def _topk_scan(groups, k, col, low_mask):
    rows = groups[0].shape[0]
    prev = jnp.full((rows, 1), -jnp.inf, dtype=jnp.float32)
    acc = jnp.zeros((rows, k), dtype=jnp.int32)
    for j in range(k):
        cands = [jnp.where(g > prev, g, jnp.inf) for g in groups]
        while len(cands) > 1:
            cands = [jnp.minimum(cands[i], cands[i + 1])
                     for i in range(0, len(cands) - 1, 2)] + (
                         [cands[-1]] if len(cands) % 2 else [])
        sel = jnp.min(cands[0], axis=-1, keepdims=True)
        sel_idx = pltpu.bitcast(sel, jnp.int32) & low_mask
        acc = jnp.where(col == j, sel_idx, acc)
        prev = sel
    return acc, prev


def _sort4(a, b, c, d):
    a, b = jnp.minimum(a, b), jnp.maximum(a, b)
    c, d = jnp.minimum(c, d), jnp.maximum(c, d)
    a, c = jnp.minimum(a, c), jnp.maximum(a, c)
    b, d = jnp.minimum(b, d), jnp.maximum(b, d)
    b, c = jnp.minimum(b, c), jnp.maximum(b, c)
    return a, b, c, d


def _edge_kernel(x_ref, out_ref, *, k, chunk):
    c, n = x_ref.shape[1], x_ref.shape[2]
    x = x_ref[0]

    norm2 = jnp.sum(x * x, axis=1, keepdims=True)
    denom = jnp.maximum(jnp.sqrt(norm2), 1e-12)
    kt = x * (1.0 / denom)
    key_sq = jnp.sum(kt * kt, axis=0, keepdims=True)
    q = jnp.transpose(kt)
    q2 = q + q

    idx_bits = max(1, (n - 1).bit_length())
    low_mask = (1 << idx_bits) - 1
    high_mask = jnp.int32(~low_mask)
    lane = jax.lax.broadcasted_iota(jnp.int32, (1, n), 1)

    out_ref[1, 0] = jax.lax.broadcasted_iota(jnp.int32, (n, k), 0)

    col = jax.lax.broadcasted_iota(jnp.int32, (chunk, k), 1)
    ngroups = n // _GW
    nchunks = n // chunk
    use_half = ngroups == 8 and k <= 4 * _GW

    gs, los, hi_mins = [], [], []
    for ci in range(nchunks):
        sl = slice(ci * chunk, (ci + 1) * chunk)
        inner2 = jnp.dot(q2[sl, :], kt,
                         preferred_element_type=jnp.float32)
        rank = key_sq - inner2
        cur = pltpu.bitcast(
            (pltpu.bitcast(rank, jnp.int32) & high_mask) | lane,
            jnp.float32)
        g = [cur[:, i * _GW:(i + 1) * _GW] for i in range(ngroups)]
        gs.append(g)
        if use_half:
            a0, a1, a2, a3 = _sort4(g[0], g[1], g[2], g[3])
            b0, b1, b2, b3 = _sort4(g[4], g[5], g[6], g[7])
            los.append([jnp.minimum(a0, b3), jnp.minimum(a1, b2),
                        jnp.minimum(a2, b1), jnp.minimum(a3, b0)])
            hi = [jnp.maximum(a0, b3), jnp.maximum(a1, b2),
                  jnp.maximum(a2, b1), jnp.maximum(a3, b0)]
            hi_mins.append(jnp.minimum(jnp.minimum(hi[0], hi[1]),
                                       jnp.minimum(hi[2], hi[3])))
        else:
            los.append(g)

    prevs = [jnp.full((chunk, 1), -jnp.inf, dtype=jnp.float32)
             for _ in range(nchunks)]
    accs = [jnp.zeros((chunk, k), dtype=jnp.int32) for _ in range(nchunks)]
    for j in range(k):
        for ci in range(nchunks):
            cands = [jnp.where(gr > prevs[ci], gr, jnp.inf)
                     for gr in los[ci]]
            while len(cands) > 1:
                cands = [jnp.minimum(cands[i], cands[i + 1])
                         for i in range(0, len(cands) - 1, 2)] + (
                             [cands[-1]] if len(cands) % 2 else [])
            sel = jnp.min(cands[0], axis=-1, keepdims=True)
            sel_idx = pltpu.bitcast(sel, jnp.int32) & low_mask
            accs[ci] = jnp.where(col == j, sel_idx, accs[ci])
            prevs[ci] = sel

    for ci in range(nchunks):
        sl = slice(ci * chunk, (ci + 1) * chunk)
        out_ref[0, 0, sl, :] = accs[ci]
        if use_half:
            bad = jnp.max(jnp.where(hi_mins[ci] < prevs[ci], 1.0, 0.0))

            @pl.when(bad > 0.0)
            def _(ci=ci, sl=sl):
                acc_full, _ = _topk_scan(gs[ci], k, col, low_mask)
                out_ref[0, 0, sl, :] = acc_full


def kernel(x):
    B, C, N, _ = x.shape
    k = _K
    x_cn = jnp.squeeze(x, -1).astype(jnp.float32)

    edge = pl.pallas_call(
        functools.partial(_edge_kernel, k=k, chunk=_CHUNK),
        out_shape=jax.ShapeDtypeStruct((2, B, N, k), jnp.int32),
        grid=(B,),
        in_specs=[
            pl.BlockSpec((1, C, N), lambda b: (b, 0, 0)),
        ],
        out_specs=pl.BlockSpec((2, 1, N, k), lambda b: (0, b, 0, 0)),
        compiler_params=pltpu.CompilerParams(
            dimension_semantics=("parallel",),
            vmem_limit_bytes=48 << 20),
    )(x_cn)
    return edge


# speedup vs baseline: 1.0549x; 1.0549x over previous
import functools

import jax
import jax.numpy as jnp
from jax.experimental import pallas as pl
from jax.experimental.pallas import tpu as pltpu

_K = 20
_CHUNK = 256
_GW = 128


def _topk_scan(groups, k, col, low_mask):
    rows = groups[0].shape[0]
    prev = jnp.full((rows, 1), -jnp.inf, dtype=jnp.float32)
    acc = jnp.zeros((rows, k), dtype=jnp.int32)
    for j in range(k):
        cands = [jnp.where(g > prev, g, jnp.inf) for g in groups]
        while len(cands) > 1:
            cands = [jnp.minimum(cands[i], cands[i + 1])
                     for i in range(0, len(cands) - 1, 2)] + (
                         [cands[-1]] if len(cands) % 2 else [])
        sel = jnp.min(cands[0], axis=-1, keepdims=True)
        sel_idx = pltpu.bitcast(sel, jnp.int32) & low_mask
        acc = jnp.where(col == j, sel_idx, acc)
        prev = sel
    return acc, prev


def _sort4(a, b, c, d):
    a, b = jnp.minimum(a, b), jnp.maximum(a, b)
    c, d = jnp.minimum(c, d), jnp.maximum(c, d)
    a, c = jnp.minimum(a, c), jnp.maximum(a, c)
    b, d = jnp.minimum(b, d), jnp.maximum(b, d)
    b, c = jnp.minimum(b, c), jnp.maximum(b, c)
    return a, b, c, d


def _edge_kernel(x_ref, out_ref, *, k, chunk):
    c, n = x_ref.shape[1], x_ref.shape[2]
    x = x_ref[0]

    norm2 = jnp.sum(x * x, axis=1, keepdims=True)
    denom = jnp.maximum(jnp.sqrt(norm2), 1e-12)
    kt = x * (1.0 / denom)
    key_sq = jnp.sum(kt * kt, axis=0, keepdims=True)
    q = jnp.transpose(kt)
    q2 = q + q

    idx_bits = max(1, (n - 1).bit_length())
    low_mask = (1 << idx_bits) - 1
    high_mask = jnp.int32(~low_mask)
    lane = jax.lax.broadcasted_iota(jnp.int32, (1, n), 1)

    out_ref[1, 0] = jax.lax.broadcasted_iota(jnp.int32, (n, k), 0)

    col = jax.lax.broadcasted_iota(jnp.int32, (chunk, k), 1)
    ngroups = n // _GW
    nchunks = n // chunk
    use_half = ngroups == 8 and k <= 4 * _GW

    gs, los, hi_mins = [], [], []
    for ci in range(nchunks):
        sl = slice(ci * chunk, (ci + 1) * chunk)
        inner2 = jnp.dot(q2[sl, :], kt,
                         preferred_element_type=jnp.float32)
        rank = key_sq - inner2
        cur = pltpu.bitcast(
            (pltpu.bitcast(rank, jnp.int32) & high_mask) | lane,
            jnp.float32)
        g = [cur[:, i * _GW:(i + 1) * _GW] for i in range(ngroups)]
        gs.append(g)
        if use_half:
            a0, a1, a2, a3 = _sort4(g[0], g[1], g[2], g[3])
            b0, b1, b2, b3 = _sort4(g[4], g[5], g[6], g[7])
            los.append([jnp.minimum(a0, b3), jnp.minimum(a1, b2),
                        jnp.minimum(a2, b1), jnp.minimum(a3, b0)])
            hi = [jnp.maximum(a0, b3), jnp.maximum(a1, b2),
                  jnp.maximum(a2, b1), jnp.maximum(a3, b0)]
            hi_mins.append(jnp.minimum(jnp.minimum(hi[0], hi[1]),
                                       jnp.minimum(hi[2], hi[3])))
        else:
            los.append(g)

    prevs = [jnp.full((chunk, 1), -jnp.inf, dtype=jnp.float32)
             for _ in range(nchunks)]
    accs = [jnp.zeros((chunk, k), dtype=jnp.int32) for _ in range(nchunks)]
    for j in range(k):
        for ci in range(nchunks):
            cands = [jnp.where(gr > prevs[ci], gr, jnp.inf)
                     for gr in los[ci]]
            while len(cands) > 1:
                cands = [jnp.minimum(cands[i], cands[i + 1])
                         for i in range(0, len(cands) - 1, 2)] + (
                             [cands[-1]] if len(cands) % 2 else [])
            sel = jnp.min(cands[0], axis=-1, keepdims=True)
            sel_idx = pltpu.bitcast(sel, jnp.int32) & low_mask
            accs[ci] = jnp.where(col == j, sel_idx, accs[ci])
            prevs[ci] = sel

    for ci in range(nchunks):
        sl = slice(ci * chunk, (ci + 1) * chunk)
        out_ref[0, 0, sl, :] = accs[ci]
        if use_half:
            bad = jnp.max(jnp.where(hi_mins[ci] < prevs[ci], 1.0, 0.0))

            @pl.when(bad > 0.0)
            def _(ci=ci, sl=sl):
                acc_full, _ = _topk_scan(gs[ci], k, col, low_mask)
                out_ref[0, 0, sl, :] = acc_full


def kernel(x):
    B, C, N, _ = x.shape
    k = _K
    x_cn = jnp.squeeze(x, -1).astype(jnp.float32)

    edge = pl.pallas_call(
        functools.partial(_edge_kernel, k=k, chunk=_CHUNK),
        out_shape=jax.ShapeDtypeStruct((2, B, N, k), jnp.int32),
        grid=(B,),
        in_specs=[
            pl.BlockSpec((1, C, N), lambda b: (b, 0, 0)),
        ],
        out_specs=pl.BlockSpec((2, 1, N, k), lambda b: (0, b, 0, 0)),
        compiler_params=pltpu.CompilerParams(
            dimension_semantics=("parallel",),
            vmem_limit_bytes=48 << 20),
    )(x_cn)
    return edge
